# Initial kernel scaffold; baseline (speedup 1.0000x reference)
import jax
import jax.numpy as jnp
from jax.experimental import pallas as pl


def kernel(x, patch_w, patch_b, router_w, router_b, ln1_s, ln1_b, q_w, q_b, k_w, k_b, v_w, v_b, o_w, o_b, ln2_s, ln2_b, l1_w, l1_b, l2_w, l2_b, head_w, head_b, alpha):
    raise NotImplementedError("write your pallas kernel here")


# speedup vs baseline: 2.2151x; 2.2151x over previous
import jax
import jax.numpy as jnp
from jax import lax
from jax.experimental import pallas as pl
from jax.experimental.pallas import tpu as pltpu

D = 256
NE = 4
NL = 8
HEADS = 8
PATCH = 4
IMG = 128
NCLS = 10
NB = 4
G = IMG // PATCH
T = G * G
R = NB * T
NTPE = T // NE
DH = D // HEADS
INNER = 4 * D
PD = 3 * PATCH * PATCH
SCALE = float(D) ** -0.5


def _dot_t(a, b):
    return lax.dot_general(a, b, (((1,), (1,)), ((), ())),
                           preferred_element_type=jnp.float32)


def _dot(a, b):
    return lax.dot_general(a, b, (((1,), (0,)), ((), ())),
                           preferred_element_type=jnp.float32)


def _ln(x, s, b):
    mu = jnp.mean(x, axis=-1, keepdims=True)
    xc = x - mu
    var = jnp.mean(xc * xc, axis=-1, keepdims=True)
    return xc / jnp.sqrt(var + 1e-5) * s + b


def _mask(rows):
    e = (lax.broadcasted_iota(jnp.int32, (rows, D), 0) // NTPE) % NE
    width = jnp.int32(D) >> e
    col = lax.broadcasted_iota(jnp.int32, (rows, D), 1)
    return (col < width).astype(jnp.float32)


def _body(xp, patch_w, patch_b, router_w, router_b,
          ln1_s, ln1_b, q_w, q_b, k_w, k_b, v_w, v_b, o_w, o_b,
          ln2_s, ln2_b, l1_w, l1_b, l2_w, l2_b, head_w, head_b, alpha,
          out, h, sf, qs, ks, vs, ao):
    l = pl.program_id(0)
    maskR = _mask(R)

    @pl.when(l == 0)
    def _init():
        tok = _dot_t(xp[...], patch_w[...]) + patch_b[...]
        h[...] = tok
        logits = _dot_t(tok, router_w[...]) + router_b[...]
        mx = jnp.max(logits, axis=-1, keepdims=True)
        p = jnp.exp(logits - mx)
        p = p / jnp.sum(p, axis=-1, keepdims=True)
        e = (lax.broadcasted_iota(jnp.int32, (R, NE), 0) // NTPE) % NE
        col = lax.broadcasted_iota(jnp.int32, (R, NE), 1)
        ep = jnp.sum(jnp.where(col == e, p, 0.0), axis=-1, keepdims=True)
        sf[...] = alpha[0, 0] * ep + 1.0

    tnm = _ln(h[...], ln1_s[0], ln1_b[0]) * maskR
    qs[...] = (_dot_t(tnm, q_w[0]) + q_b[0]) * SCALE
    ks[...] = _dot_t(tnm, k_w[0]) + k_b[0]
    vs[...] = _dot_t(tnm, v_w[0]) + v_b[0]
    for b in range(NB):
        r0 = b * T
        for hd in range(HEADS):
            c0 = hd * DH
            s = _dot_t(qs[r0:r0 + T, c0:c0 + DH], ks[r0:r0 + T, c0:c0 + DH])
            s = s - jnp.max(s, axis=-1, keepdims=True)
            p = jnp.exp(s)
            p = p / jnp.sum(p, axis=-1, keepdims=True)
            ao[r0:r0 + T, c0:c0 + DH] = _dot(p, vs[r0:r0 + T, c0:c0 + DH])
    proj = (_dot_t(ao[...] * maskR, o_w[0]) + o_b[0]) * maskR
    h[...] = h[...] + proj

    maskT = _mask(T)
    for c in range(NB):
        r0 = c * T
        hc = h[r0:r0 + T, :]
        tn2 = _ln(hc, ln2_s[0], ln2_b[0]) * maskT
        pre = _dot_t(tn2, l1_w[0]) + l1_b[0]
        ip = pre * 0.5 * (1.0 + lax.erf(pre * (2.0 ** -0.5)))
        op = (_dot_t(ip, l2_w[0]) + l2_b[0]) * maskT
        h[r0:r0 + T, :] = hc + sf[r0:r0 + T, :] * op

    @pl.when(l == NL - 1)
    def _fin():
        pooled = jnp.concatenate(
            [jnp.mean(h[b * T:(b + 1) * T, :], axis=0, keepdims=True)
             for b in range(NB)], axis=0)
        out[...] = _dot_t(pooled, head_w[...]) + head_b[...]


def kernel(x, patch_w, patch_b, router_w, router_b, ln1_s, ln1_b,
           q_w, q_b, k_w, k_b, v_w, v_b, o_w, o_b, ln2_s, ln2_b,
           l1_w, l1_b, l2_w, l2_b, head_w, head_b, alpha):
    xp = (x.reshape(NB, 3, G, PATCH, G, PATCH)
          .transpose(0, 2, 4, 1, 3, 5).reshape(R, PD))

    c0 = lambda l: (0, 0)
    per_l2 = lambda l: (l, 0)
    per_l3 = lambda l: (l, 0, 0)
    in_specs = [
        pl.BlockSpec((R, PD), c0),
        pl.BlockSpec((D, PD), c0),
        pl.BlockSpec((1, D), c0),
        pl.BlockSpec((NE, D), c0),
        pl.BlockSpec((1, NE), c0),
        pl.BlockSpec((1, 1, D), per_l3),
        pl.BlockSpec((1, 1, D), per_l3),
        pl.BlockSpec((1, D, D), per_l3),
        pl.BlockSpec((1, 1, D), per_l3),
        pl.BlockSpec((1, D, D), per_l3),
        pl.BlockSpec((1, 1, D), per_l3),
        pl.BlockSpec((1, D, D), per_l3),
        pl.BlockSpec((1, 1, D), per_l3),
        pl.BlockSpec((1, D, D), per_l3),
        pl.BlockSpec((1, 1, D), per_l3),
        pl.BlockSpec((1, 1, D), per_l3),
        pl.BlockSpec((1, 1, D), per_l3),
        pl.BlockSpec((1, INNER, D), per_l3),
        pl.BlockSpec((1, 1, INNER), per_l3),
        pl.BlockSpec((1, D, INNER), per_l3),
        pl.BlockSpec((1, 1, D), per_l3),
        pl.BlockSpec((NCLS, D), c0),
        pl.BlockSpec((1, NCLS), c0),
        pl.BlockSpec((1, 1), c0),
    ]
    return pl.pallas_call(
        _body,
        grid=(NL,),
        in_specs=in_specs,
        out_specs=pl.BlockSpec((NB, NCLS), c0),
        out_shape=jax.ShapeDtypeStruct((NB, NCLS), jnp.float32),
        scratch_shapes=[
            pltpu.VMEM((R, D), jnp.float32),
            pltpu.VMEM((R, 1), jnp.float32),
            pltpu.VMEM((R, D), jnp.float32),
            pltpu.VMEM((R, D), jnp.float32),
            pltpu.VMEM((R, D), jnp.float32),
            pltpu.VMEM((R, D), jnp.float32),
        ],
        compiler_params=pltpu.CompilerParams(
            dimension_semantics=("arbitrary",)),
    )(xp, patch_w, patch_b.reshape(1, D), router_w, router_b.reshape(1, NE),
      ln1_s.reshape(NL, 1, D), ln1_b.reshape(NL, 1, D),
      q_w, q_b.reshape(NL, 1, D), k_w, k_b.reshape(NL, 1, D),
      v_w, v_b.reshape(NL, 1, D), o_w, o_b.reshape(NL, 1, D),
      ln2_s.reshape(NL, 1, D), ln2_b.reshape(NL, 1, D),
      l1_w, l1_b.reshape(NL, 1, INNER), l2_w, l2_b.reshape(NL, 1, D),
      head_w, head_b.reshape(1, NCLS), alpha.reshape(1, 1))
